# baseline (speedup 1.0000x reference)
import functools

import jax
import jax.numpy as jnp
from jax import lax
from jax.experimental import pallas as pl
from jax.experimental.pallas import tpu as pltpu
from jax.experimental.pallas import tpu_sc as plsc

_NC = 2
_NS = 16
_NW = _NC * _NS
_L = 16

_CHUNK = 128
_CB = 8


def _edge_transform2(edge_attr, W1, b1, W2, b2, E_pad):
    E, DE = edge_attr.shape
    D = W1.shape[1]
    BE = 1280
    assert E % BE == 0 and E_pad % BE == 0

    def body(a_ref, w1_ref, b1_ref, w2_ref, b2_ref, e1_ref, e2_ref):
        a = a_ref[...]
        e1_ref[...] = (
            jnp.dot(a, w1_ref[...], preferred_element_type=jnp.float32)
            + b1_ref[...]
        )
        e2_ref[...] = (
            jnp.dot(a, w2_ref[...], preferred_element_type=jnp.float32)
            + b2_ref[...]
        )

    wspec = pl.BlockSpec((DE, D), lambda i: (0, 0))
    bspec = pl.BlockSpec((1, D), lambda i: (0, 0))
    return pl.pallas_call(
        body,
        grid=(E // BE,),
        in_specs=[pl.BlockSpec((BE, DE), lambda i: (i, 0)),
                  wspec, bspec, wspec, bspec],
        out_specs=[
            pl.BlockSpec((BE, D), lambda i: (i, 0)),
            pl.BlockSpec((BE, D), lambda i: (i, 0)),
        ],
        out_shape=[
            jax.ShapeDtypeStruct((E_pad, D), jnp.float32),
            jax.ShapeDtypeStruct((E_pad, D), jnp.float32),
        ],
    )(edge_attr, W1.astype(jnp.float32), b1.reshape(1, D),
      W2.astype(jnp.float32), b2.reshape(1, D))


def _sc_message_aggregate(x, src2d, dst2d, e, n_pad):
    D = x.shape[1]
    n_chunks = e.shape[0] // _CHUNK
    npt = n_pad // _NS
    assert n_chunks % (_NS * _CB) == 0 and n_pad % (_NS * _CHUNK) == 0
    hops = npt // _CHUNK
    cpw = n_chunks // _NW
    assert cpw % _CB == 0

    mesh = plsc.VectorSubcoreMesh(core_axis_name="c", subcore_axis_name="s")

    @functools.partial(
        pl.kernel,
        out_type=jax.ShapeDtypeStruct((_NC, n_pad, D), jnp.float32),
        mesh=mesh,
        scratch_types=[
            pltpu.VMEM((_CB, _CHUNK), jnp.int32),
            pltpu.VMEM((_CB, _CHUNK), jnp.int32),
            pltpu.VMEM((_CHUNK, D), jnp.float32),
            pltpu.VMEM((_CHUNK, D), jnp.float32),
            pltpu.VMEM((_CHUNK // 2, D), jnp.float32),
            pltpu.VMEM_SHARED((n_pad, D), jnp.float32),
            pltpu.SemaphoreType.DMA,
        ],
    )
    def k(x_hbm, src_hbm, dst_hbm, e_hbm, out_hbm, srcv, dstv, xrows0,
          xrows1, erows, acc, gsem):
        c = lax.axis_index("c")
        s = lax.axis_index("s")

        def zrow(i, carry):
            for t in range(D // _L):
                xrows0[i, pl.ds(t * _L, _L)] = jnp.zeros((_L,), jnp.float32)
            return carry
        lax.fori_loop(0, _CHUNK, zrow, 0)
        for h in range(hops):
            r0 = pl.multiple_of(s * npt + h * _CHUNK, _CHUNK)
            pltpu.sync_copy(xrows0, acc.at[pl.ds(r0, _CHUNK)])
        plsc.subcore_barrier()

        def chunk_base(j):
            block = 2 * (_NS * lax.div(j, _CB) + s) + c
            return pl.multiple_of(block * _CB, _CB)

        def issue(j, xbuf):
            cb = lax.rem(j, _CB)

            @pl.when(cb == 0)
            def _load_src():
                pltpu.sync_copy(src_hbm.at[pl.ds(chunk_base(j), _CB)], srcv)

            pltpu.async_copy(x_hbm.at[srcv.at[cb]], xbuf, gsem)

        def step(j, xbuf, nxbuf):
            cb = lax.rem(j, _CB)

            @pl.when(cb == 0)
            def _load_dst():
                pltpu.sync_copy(dst_hbm.at[pl.ds(chunk_base(j), _CB)], dstv)

            pltpu.make_async_copy(x_hbm.at[srcv.at[0]], xbuf, gsem).wait()

            @pl.when(j + 1 < cpw)
            def _issue_next():
                issue(j + 1, nxbuf)

            hc = _CHUNK // 2
            for half in range(2):
                ebase = pl.multiple_of(
                    (chunk_base(j) + cb) * _CHUNK + half * hc, hc)
                pltpu.sync_copy(e_hbm.at[pl.ds(ebase, hc)], erows)

                def row(i, rcarry):
                    for t in range(D // _L):
                        sl = pl.ds(t * _L, _L)
                        xi = half * hc + i
                        xbuf[xi, sl] = jnp.maximum(
                            xbuf[xi, sl] + erows[i, sl], 0.0)
                    return rcarry
                lax.fori_loop(0, hc, row, 0)

            pltpu.sync_copy(xbuf, acc.at[dstv.at[cb]], add=True)

        issue(0, xrows0)

        def pair_body(t, carry):
            step(2 * t, xrows0, xrows1)
            step(2 * t + 1, xrows1, xrows0)
            return carry

        lax.fori_loop(0, cpw // 2, pair_body, 0)
        plsc.subcore_barrier()

        for h in range(hops):
            r0 = pl.multiple_of(s * npt + h * _CHUNK, _CHUNK)
            pltpu.sync_copy(acc.at[pl.ds(r0, _CHUNK)],
                            out_hbm.at[c, pl.ds(r0, _CHUNK)])

    return k(x, src2d, dst2d, e)


def _node_mlp(x, partials, Wa, ba, g, be, Wb, bb):
    N, D = x.shape
    H = Wa.shape[1]
    n_pad = partials.shape[1]

    def body(x_ref, p_ref, wa_ref, ba_ref, g_ref, be_ref, wb_ref, bb_ref,
             o_ref):
        z = x_ref[...] + p_ref[0, :N, :] + p_ref[1, :N, :]
        h = jnp.dot(z, wa_ref[...], preferred_element_type=jnp.float32)
        h = h + ba_ref[...]
        m = jnp.mean(h, axis=0, keepdims=True)
        v = jnp.mean(jnp.square(h - m), axis=0, keepdims=True)
        h = (h - m) * jax.lax.rsqrt(v + 1e-5) * g_ref[...] + be_ref[...]
        h = jnp.maximum(h, 0.0)
        h = jnp.dot(h, wb_ref[...], preferred_element_type=jnp.float32)
        o_ref[...] = jnp.maximum(h + bb_ref[...], 0.0)

    return pl.pallas_call(
        body,
        out_shape=jax.ShapeDtypeStruct((N, H), jnp.float32),
    )(x, partials, Wa, ba.reshape(1, H), g.reshape(1, H), be.reshape(1, H),
      Wb, bb.reshape(1, H))


def _readout(h, Wr_pad, br_pad, WeT_pad, bee):
    N, H = h.shape
    R = Wr_pad.shape[1]
    BN_ = 2000
    assert N % BN_ == 0

    def body(h_ref, wr_ref, br_ref, we_ref, bee_ref, o_ref):
        r = jnp.dot(h_ref[...], wr_ref[...],
                    preferred_element_type=jnp.float32) + br_ref[...]
        r = jnp.where(r > 0, r, 0.01 * r)
        o_ref[...] = (jnp.sum(r * we_ref[...], axis=1, keepdims=True)
                      + bee_ref[...])

    return pl.pallas_call(
        body,
        grid=(N // BN_,),
        in_specs=[
            pl.BlockSpec((BN_, H), lambda i: (i, 0)),
            pl.BlockSpec((H, R), lambda i: (0, 0)),
            pl.BlockSpec((1, R), lambda i: (0, 0)),
            pl.BlockSpec((1, R), lambda i: (0, 0)),
            pl.BlockSpec((1, 1), lambda i: (0, 0)),
        ],
        out_specs=pl.BlockSpec((BN_, 1), lambda i: (i, 0)),
        out_shape=jax.ShapeDtypeStruct((N, 1), jnp.float32),
    )(h, Wr_pad, br_pad, WeT_pad, bee.reshape(1, 1))


def kernel(x, edge_index, edge_attr, batch, epoch, lin_e1_W, lin_e1_b, W1a,
           b1a, g1, be1, W1b, b1b, lin_e2_W, lin_e2_b, W2a, b2a, g2, be2,
           W2b, b2b, Wr, br, We, bee):
    N, D = x.shape
    E = edge_index.shape[1]
    H = W1a.shape[1]
    R = Wr.shape[1]

    e_align = _NW * _CHUNK * _CB
    E_pad = ((E + e_align - 1) // e_align) * e_align
    pad = E_pad - E
    src = edge_index[0]
    dst = edge_index[1]
    if pad:
        src = jnp.concatenate([src, jnp.zeros((pad,), jnp.int32)])
        dst = jnp.concatenate([dst, jnp.full((pad,), N, jnp.int32)])
    src2d = src.reshape(E_pad // _CHUNK, _CHUNK)
    dst2d = dst.reshape(E_pad // _CHUNK, _CHUNK)

    n_align = _NS * _CHUNK
    n_pad = ((N + n_align - 1) // n_align) * n_align

    e1, e2 = _edge_transform2(edge_attr, lin_e1_W, lin_e1_b, lin_e2_W,
                              lin_e2_b, E_pad)
    p1 = _sc_message_aggregate(x, src2d, dst2d, e1, n_pad)
    h1 = _node_mlp(x, p1, W1a, b1a, g1, be1, W1b, b1b)

    p2 = _sc_message_aggregate(h1, src2d, dst2d, e2, n_pad)
    h2 = _node_mlp(h1, p2, W2a, b2a, g2, be2, W2b, b2b)

    R_pad = ((R + 127) // 128) * 128
    Wr_pad = jnp.pad(Wr, ((0, 0), (0, R_pad - R)))
    br_pad = jnp.pad(br, (0, R_pad - R)).reshape(1, R_pad)
    WeT_pad = jnp.pad(We[:, 0], (0, R_pad - R)).reshape(1, R_pad)

    return _readout(h2, Wr_pad, br_pad, WeT_pad, bee)


# speedup vs baseline: 1.0098x; 1.0098x over previous
import functools

import jax
import jax.numpy as jnp
from jax import lax
from jax.experimental import pallas as pl
from jax.experimental.pallas import tpu as pltpu
from jax.experimental.pallas import tpu_sc as plsc

_NC = 2
_NS = 16
_NW = _NC * _NS
_L = 16

_CHUNK = 128
_CB = 8


def _edge_transform2(edge_attr, W1, b1, W2, b2):
    E, DE = edge_attr.shape
    D = W1.shape[1]
    BE = 2048
    assert E % BE == 0

    def body(a_ref, w1_ref, b1_ref, w2_ref, b2_ref, e1_ref, e2_ref):
        a = a_ref[...]
        e1_ref[...] = (
            jnp.dot(a, w1_ref[...], preferred_element_type=jnp.float32)
            + b1_ref[...]
        )
        e2_ref[...] = (
            jnp.dot(a, w2_ref[...], preferred_element_type=jnp.float32)
            + b2_ref[...]
        )

    wspec = pl.BlockSpec((DE, D), lambda i: (0, 0))
    bspec = pl.BlockSpec((1, D), lambda i: (0, 0))
    return pl.pallas_call(
        body,
        grid=(E // BE,),
        in_specs=[pl.BlockSpec((BE, DE), lambda i: (i, 0)),
                  wspec, bspec, wspec, bspec],
        out_specs=[
            pl.BlockSpec((BE, D), lambda i: (i, 0)),
            pl.BlockSpec((BE, D), lambda i: (i, 0)),
        ],
        out_shape=[
            jax.ShapeDtypeStruct((E, D), jnp.float32),
            jax.ShapeDtypeStruct((E, D), jnp.float32),
        ],
    )(edge_attr, W1.astype(jnp.float32), b1.reshape(1, D),
      W2.astype(jnp.float32), b2.reshape(1, D))


def _sc_message_aggregate(x, src2d, dst2d, e, n_pad):
    D = x.shape[1]
    n_chunks = e.shape[0] // _CHUNK
    npt = n_pad // _NS
    assert n_chunks % (_NS * _CB) == 0 and n_pad % (_NS * _CHUNK) == 0
    hops = npt // _CHUNK
    cpw = n_chunks // _NW
    assert cpw % _CB == 0

    mesh = plsc.VectorSubcoreMesh(core_axis_name="c", subcore_axis_name="s")

    @functools.partial(
        pl.kernel,
        out_type=jax.ShapeDtypeStruct((_NC, n_pad, D), jnp.float32),
        mesh=mesh,
        scratch_types=[
            pltpu.VMEM((_CB, _CHUNK), jnp.int32),
            pltpu.VMEM((_CB, _CHUNK), jnp.int32),
            pltpu.VMEM((_CHUNK, D), jnp.float32),
            pltpu.VMEM((_CHUNK, D), jnp.float32),
            pltpu.VMEM((_CHUNK // 2, D), jnp.float32),
            pltpu.VMEM_SHARED((n_pad, D), jnp.float32),
            pltpu.SemaphoreType.DMA,
        ],
    )
    def k(x_hbm, src_hbm, dst_hbm, e_hbm, out_hbm, srcv, dstv, xrows0,
          xrows1, erows, acc, gsem):
        c = lax.axis_index("c")
        s = lax.axis_index("s")

        def zrow(i, carry):
            for t in range(D // _L):
                xrows0[i, pl.ds(t * _L, _L)] = jnp.zeros((_L,), jnp.float32)
            return carry
        lax.fori_loop(0, _CHUNK, zrow, 0)
        for h in range(hops):
            r0 = pl.multiple_of(s * npt + h * _CHUNK, _CHUNK)
            pltpu.sync_copy(xrows0, acc.at[pl.ds(r0, _CHUNK)])
        plsc.subcore_barrier()

        def chunk_base(j):
            block = 2 * (_NS * lax.div(j, _CB) + s) + c
            return pl.multiple_of(block * _CB, _CB)

        def issue(j, xbuf):
            cb = lax.rem(j, _CB)

            @pl.when(cb == 0)
            def _load_src():
                pltpu.sync_copy(src_hbm.at[pl.ds(chunk_base(j), _CB)], srcv)

            pltpu.async_copy(x_hbm.at[srcv.at[cb]], xbuf, gsem)

        def step(j, xbuf, nxbuf):
            cb = lax.rem(j, _CB)

            @pl.when(cb == 0)
            def _load_dst():
                pltpu.sync_copy(dst_hbm.at[pl.ds(chunk_base(j), _CB)], dstv)

            pltpu.make_async_copy(x_hbm.at[srcv.at[0]], xbuf, gsem).wait()

            @pl.when(j + 1 < cpw)
            def _issue_next():
                issue(j + 1, nxbuf)

            hc = _CHUNK // 2
            for half in range(2):
                ebase = pl.multiple_of(
                    (chunk_base(j) + cb) * _CHUNK + half * hc, hc)
                pltpu.sync_copy(e_hbm.at[pl.ds(ebase, hc)], erows)

                def row(i, rcarry):
                    for t in range(D // _L):
                        sl = pl.ds(t * _L, _L)
                        xi = half * hc + i
                        xbuf[xi, sl] = jnp.maximum(
                            xbuf[xi, sl] + erows[i, sl], 0.0)
                    return rcarry
                lax.fori_loop(0, hc, row, 0)

            pltpu.sync_copy(xbuf, acc.at[dstv.at[cb]], add=True)

        issue(0, xrows0)

        def pair_body(t, carry):
            step(2 * t, xrows0, xrows1)
            step(2 * t + 1, xrows1, xrows0)
            return carry

        lax.fori_loop(0, cpw // 2, pair_body, 0)
        plsc.subcore_barrier()

        for h in range(hops):
            r0 = pl.multiple_of(s * npt + h * _CHUNK, _CHUNK)
            pltpu.sync_copy(acc.at[pl.ds(r0, _CHUNK)],
                            out_hbm.at[c, pl.ds(r0, _CHUNK)])

    return k(x, src2d, dst2d, e)


def _node_mlp(x, partials, Wa, ba, g, be, Wb, bb):
    N, D = x.shape
    H = Wa.shape[1]
    n_pad = partials.shape[1]

    def body(x_ref, p_ref, wa_ref, ba_ref, g_ref, be_ref, wb_ref, bb_ref,
             o_ref):
        z = x_ref[...] + p_ref[0, :N, :] + p_ref[1, :N, :]
        h = jnp.dot(z, wa_ref[...], preferred_element_type=jnp.float32)
        h = h + ba_ref[...]
        m = jnp.mean(h, axis=0, keepdims=True)
        v = jnp.mean(jnp.square(h - m), axis=0, keepdims=True)
        h = (h - m) * jax.lax.rsqrt(v + 1e-5) * g_ref[...] + be_ref[...]
        h = jnp.maximum(h, 0.0)
        h = jnp.dot(h, wb_ref[...], preferred_element_type=jnp.float32)
        o_ref[...] = jnp.maximum(h + bb_ref[...], 0.0)

    return pl.pallas_call(
        body,
        out_shape=jax.ShapeDtypeStruct((N, H), jnp.float32),
    )(x, partials, Wa, ba.reshape(1, H), g.reshape(1, H), be.reshape(1, H),
      Wb, bb.reshape(1, H))


def _readout(h, Wr_pad, br_pad, WeT_pad, bee):
    N, H = h.shape
    R = Wr_pad.shape[1]
    BN_ = 2000
    assert N % BN_ == 0

    def body(h_ref, wr_ref, br_ref, we_ref, bee_ref, o_ref):
        r = jnp.dot(h_ref[...], wr_ref[...],
                    preferred_element_type=jnp.float32) + br_ref[...]
        r = jnp.where(r > 0, r, 0.01 * r)
        o_ref[...] = (jnp.sum(r * we_ref[...], axis=1, keepdims=True)
                      + bee_ref[...])

    return pl.pallas_call(
        body,
        grid=(N // BN_,),
        in_specs=[
            pl.BlockSpec((BN_, H), lambda i: (i, 0)),
            pl.BlockSpec((H, R), lambda i: (0, 0)),
            pl.BlockSpec((1, R), lambda i: (0, 0)),
            pl.BlockSpec((1, R), lambda i: (0, 0)),
            pl.BlockSpec((1, 1), lambda i: (0, 0)),
        ],
        out_specs=pl.BlockSpec((BN_, 1), lambda i: (i, 0)),
        out_shape=jax.ShapeDtypeStruct((N, 1), jnp.float32),
    )(h, Wr_pad, br_pad, WeT_pad, bee.reshape(1, 1))


def kernel(x, edge_index, edge_attr, batch, epoch, lin_e1_W, lin_e1_b, W1a,
           b1a, g1, be1, W1b, b1b, lin_e2_W, lin_e2_b, W2a, b2a, g2, be2,
           W2b, b2b, Wr, br, We, bee):
    N, D = x.shape
    E = edge_index.shape[1]
    H = W1a.shape[1]
    R = Wr.shape[1]

    e_align = _NW * _CHUNK * _CB
    E_pad = ((E + e_align - 1) // e_align) * e_align
    pad = E_pad - E
    src = edge_index[0]
    dst = edge_index[1]
    if pad:
        src = jnp.concatenate([src, jnp.zeros((pad,), jnp.int32)])
        dst = jnp.concatenate([dst, jnp.full((pad,), N, jnp.int32)])
        edge_attr = jnp.concatenate(
            [edge_attr, jnp.zeros((pad, edge_attr.shape[1]),
                                  edge_attr.dtype)])
    src2d = src.reshape(E_pad // _CHUNK, _CHUNK)
    dst2d = dst.reshape(E_pad // _CHUNK, _CHUNK)

    n_align = _NS * _CHUNK
    n_pad = ((N + n_align - 1) // n_align) * n_align

    e1, e2 = _edge_transform2(edge_attr, lin_e1_W, lin_e1_b, lin_e2_W,
                              lin_e2_b)
    p1 = _sc_message_aggregate(x, src2d, dst2d, e1, n_pad)
    h1 = _node_mlp(x, p1, W1a, b1a, g1, be1, W1b, b1b)

    p2 = _sc_message_aggregate(h1, src2d, dst2d, e2, n_pad)
    h2 = _node_mlp(h1, p2, W2a, b2a, g2, be2, W2b, b2b)

    R_pad = ((R + 127) // 128) * 128
    Wr_pad = jnp.pad(Wr, ((0, 0), (0, R_pad - R)))
    br_pad = jnp.pad(br, (0, R_pad - R)).reshape(1, R_pad)
    WeT_pad = jnp.pad(We[:, 0], (0, R_pad - R)).reshape(1, R_pad)

    return _readout(h2, Wr_pad, br_pad, WeT_pad, bee)
